# baseline (device time: 262924 ns/iter reference)
import jax
import jax.numpy as jnp
from jax import lax
from jax.experimental import pallas as pl
from jax.experimental.pallas import tpu as pltpu

N_DEV = 16
T_CORR = 48


def kernel(x, A, B, C):
    b, s, d = x.shape
    n = A.shape[1]

    xt = jnp.transpose(x, (1, 0, 2))
    Bt = jnp.transpose(B, (1, 0, 2))
    Ct = jnp.transpose(C, (1, 0, 2))
    dA = jnp.exp(A.T)

    def body(x_ref, dA_ref, B_ref, C_ref, out_ref,
             h_ref, recv_ref, send_sem, recv_sem):
        my_i = lax.axis_index("i")
        left = (my_i - 1) % N_DEV
        right = (my_i + 1) % N_DEV

        barrier_sem = pltpu.get_barrier_semaphore()
        for nbr in (left, right):
            pl.semaphore_signal(
                barrier_sem, inc=1,
                device_id=(nbr,), device_id_type=pl.DeviceIdType.MESH,
            )
        pl.semaphore_wait(barrier_sem, 2)

        dAv = dA_ref[...][None]
        h_ref[...] = jnp.zeros_like(h_ref)

        def step(t, carry):
            xb = x_ref[t]
            Bb = B_ref[t]
            Cb = C_ref[t]
            h = h_ref[...] * dAv + xb[:, None, :] * Bb[:, :, None]
            h_ref[...] = h
            out_ref[t] = jnp.sum(h * Cb[:, :, None], axis=1)
            return carry

        lax.fori_loop(0, s, step, 0)

        rdma = pltpu.make_async_remote_copy(
            src_ref=h_ref,
            dst_ref=recv_ref,
            send_sem=send_sem,
            recv_sem=recv_sem,
            device_id=(right,),
            device_id_type=pl.DeviceIdType.MESH,
        )
        rdma.start()
        rdma.wait()

        @pl.when(my_i != 0)
        def _():
            def corr(t, carry):
                g = recv_ref[...] * dAv
                recv_ref[...] = g
                out_ref[t] = out_ref[t] + jnp.sum(
                    g * C_ref[t][:, :, None], axis=1)
                return carry

            lax.fori_loop(0, T_CORR, corr, 0)

    out = pl.pallas_call(
        body,
        out_shape=jax.ShapeDtypeStruct((s, b, d), jnp.float32),
        in_specs=[pl.BlockSpec(memory_space=pltpu.VMEM)] * 4,
        out_specs=pl.BlockSpec(memory_space=pltpu.VMEM),
        scratch_shapes=[
            pltpu.VMEM((b, n, d), jnp.float32),
            pltpu.VMEM((b, n, d), jnp.float32),
            pltpu.SemaphoreType.DMA,
            pltpu.SemaphoreType.DMA,
        ],
        compiler_params=pltpu.CompilerParams(collective_id=0),
    )(xt, dA, Bt, Ct)

    return jnp.transpose(out, (1, 0, 2))


# device time: 262082 ns/iter; 1.0032x vs baseline; 1.0032x over previous
import jax
import jax.numpy as jnp
from jax import lax
from jax.experimental import pallas as pl
from jax.experimental.pallas import tpu as pltpu

N_DEV = 16
T_CORR = 48


def kernel(x, A, B, C):
    b, s, d = x.shape
    n = A.shape[1]

    cdt = jnp.bfloat16
    xt = jnp.transpose(x, (1, 0, 2)).astype(cdt)
    Bt = jnp.transpose(B, (1, 0, 2)).astype(cdt)
    Ct = jnp.transpose(C, (1, 0, 2)).astype(cdt)
    dA = jnp.exp(A.T).astype(cdt)

    def body(x_ref, dA_ref, B_ref, C_ref, out_ref,
             h_ref, recv_ref, send_sem, recv_sem):
        my_i = lax.axis_index("i")
        left = (my_i - 1) % N_DEV
        right = (my_i + 1) % N_DEV

        barrier_sem = pltpu.get_barrier_semaphore()
        for nbr in (left, right):
            pl.semaphore_signal(
                barrier_sem, inc=1,
                device_id=(nbr,), device_id_type=pl.DeviceIdType.MESH,
            )
        pl.semaphore_wait(barrier_sem, 2)

        dAv = dA_ref[...][None]
        h_ref[...] = jnp.zeros_like(h_ref)

        def step(t, carry):
            xb = x_ref[t]
            Bb = B_ref[t]
            Cb = C_ref[t]
            h = h_ref[...] * dAv + xb[:, None, :] * Bb[:, :, None]
            h_ref[...] = h
            out_ref[t] = jnp.sum(
                h * Cb[:, :, None], axis=1, dtype=jnp.float32)
            return carry

        lax.fori_loop(0, s, step, 0)

        rdma = pltpu.make_async_remote_copy(
            src_ref=h_ref,
            dst_ref=recv_ref,
            send_sem=send_sem,
            recv_sem=recv_sem,
            device_id=(right,),
            device_id_type=pl.DeviceIdType.MESH,
        )
        rdma.start()
        rdma.wait()

        @pl.when(my_i != 0)
        def _():
            def corr(t, carry):
                g = recv_ref[...] * dAv
                recv_ref[...] = g
                out_ref[t] = out_ref[t] + jnp.sum(
                    g * C_ref[t][:, :, None], axis=1, dtype=jnp.float32)
                return carry

            lax.fori_loop(0, T_CORR, corr, 0)

    out = pl.pallas_call(
        body,
        out_shape=jax.ShapeDtypeStruct((s, b, d), jnp.float32),
        in_specs=[pl.BlockSpec(memory_space=pltpu.VMEM)] * 4,
        out_specs=pl.BlockSpec(memory_space=pltpu.VMEM),
        scratch_shapes=[
            pltpu.VMEM((b, n, d), jnp.bfloat16),
            pltpu.VMEM((b, n, d), jnp.bfloat16),
            pltpu.SemaphoreType.DMA,
            pltpu.SemaphoreType.DMA,
        ],
        compiler_params=pltpu.CompilerParams(collective_id=0),
    )(xt, dA, Bt, Ct)

    return jnp.transpose(out, (1, 0, 2))
